# baseline (speedup 1.0000x reference)
import functools

import jax
import jax.numpy as jnp
from jax import lax
from jax.experimental import pallas as pl
from jax.experimental.pallas import tpu as pltpu
from jax.experimental.pallas import tpu_sc as plsc

MAX_NORM = 1.0
BN = 3584
DPAD = 384


@functools.lru_cache(maxsize=None)
def _make_sc_gather(V, D, B):
    info = plsc.get_sparse_core_info()
    NC, NS = info.num_cores, info.num_subcores
    NW = NC * NS
    assert B % NW == 0
    b_per_w = B // NW
    mesh = plsc.VectorSubcoreMesh(core_axis_name="c", subcore_axis_name="s")

    @functools.partial(
        pl.kernel,
        mesh=mesh,
        out_type=jax.ShapeDtypeStruct((B, DPAD), jnp.float32),
        scratch_types=[
            pltpu.VMEM((b_per_w,), jnp.int32),
            pltpu.VMEM((b_per_w, 256), jnp.float32),
            pltpu.VMEM((b_per_w, 128), jnp.float32),
            pltpu.SemaphoreType.DMA,
        ],
        compiler_params=pltpu.CompilerParams(disable_bounds_checks=True),
    )
    def sc_gather(idx_hbm, table_hbm, out_hbm, idx_v, ca, cb, sem):
        wid = lax.axis_index("s") * NC + lax.axis_index("c")
        base = wid * b_per_w
        pltpu.sync_copy(idx_hbm.at[pl.ds(base, b_per_w)], idx_v)
        cpa = pltpu.async_copy(table_hbm.at[idx_v, pl.ds(0, 256)], ca, sem)
        tail_start = pl.multiple_of(jnp.int32(2 * 128), 128)
        cpb = pltpu.async_copy(table_hbm.at[idx_v, pl.ds(tail_start, 128)], cb, sem)
        cpa.wait()
        cpb.wait()
        pltpu.sync_copy(ca, out_hbm.at[pl.ds(base, b_per_w), pl.ds(0, 256)])
        pltpu.sync_copy(cb, out_hbm.at[pl.ds(base, b_per_w), pl.ds(256, 128)])

    return sc_gather


def _proj_body(emb_ref, wt_ref, b_ref, out_ref, es_ref):
    @pl.when(pl.program_id(0) == 0)
    def _():
        emb = emb_ref[:, :300]
        sumsq = jnp.sum(emb * emb, axis=1, keepdims=True)
        norm = jnp.sqrt(sumsq)
        scale = jnp.minimum(1.0, MAX_NORM / jnp.maximum(norm, 1e-7))
        es_ref[...] = emb * scale

    acc = lax.dot_general(
        wt_ref[...], es_ref[...], (((0,), (1,)), ((), ())),
        preferred_element_type=jnp.float32,
    )
    j = pl.program_id(0)
    lane = lax.broadcasted_iota(jnp.int32, (1, b_ref.shape[1]), 1)
    bcol = jnp.sum(
        jnp.where(lane == j, b_ref[...], 0.0), axis=1, keepdims=True
    )
    out_ref[...] = acc + bcol


def _projection(emb, Wt, b2):
    B = emb.shape[0]
    D, V = Wt.shape
    grid = (pl.cdiv(V, BN),)
    outT = pl.pallas_call(
        _proj_body,
        grid=grid,
        in_specs=[
            pl.BlockSpec((B, DPAD), lambda j: (0, 0)),
            pl.BlockSpec((D, BN), lambda j: (0, j)),
            pl.BlockSpec((BN, 128), lambda j: (0, 0)),
        ],
        out_specs=pl.BlockSpec((BN, B), lambda j: (j, 0)),
        out_shape=jax.ShapeDtypeStruct((V, B), jnp.float32),
        scratch_shapes=[pltpu.VMEM((B, D), jnp.float32)],
    )(emb, Wt, b2)
    return outT.T


def kernel(inputs_, table, W, b):
    V, D = table.shape
    B = inputs_.shape[0]
    emb = _make_sc_gather(V, D, B)(inputs_, table)
    nt = pl.cdiv(V, BN)
    bp = jnp.pad(b, (0, nt * BN - V)).reshape(nt, BN)
    b2 = jnp.pad(bp, ((0, 128 - nt), (0, 0))).T
    return _projection(emb, W.T, b2)
